# baseline (speedup 1.0000x reference)
import functools

import jax
import jax.numpy as jnp
from jax import lax
from jax.experimental import pallas as pl
from jax.experimental.pallas import tpu as pltpu
from jax.experimental.pallas import tpu_sc as plsc

N = 10000
E = 320000
H = 128
NPAD = 10240
NC = 2
NS = 16
NW = NC * NS
CK = 112
CPW = 93
EPW = CPW * CK
EP = NW * EPW
RPT = NPAD // NS


def _tc_prep_body(x_ref, w_ref, asr_ref, adr_ref, h_ref, asad_ref, gm_ref):
    h = jnp.dot(x_ref[...], w_ref[...].T, preferred_element_type=jnp.float32)
    h_ref[...] = h
    a_s = jnp.sum(h * asr_ref[...][None, :], axis=1)
    a_d = jnp.sum(h * adr_ref[...][None, :], axis=1)
    asad_ref[...] = jnp.stack([a_s, a_d])
    m = jnp.max(a_s) + jnp.max(a_d)
    m = jnp.where(m >= 0.0, m, 0.2 * m)
    gm_ref[...] = jnp.full((16,), m, jnp.float32)


_tc_prep = pl.pallas_call(
    _tc_prep_body,
    out_shape=(
        jax.ShapeDtypeStruct((NPAD, H), jnp.float32),
        jax.ShapeDtypeStruct((2, NPAD), jnp.float32),
        jax.ShapeDtypeStruct((16,), jnp.float32),
    ),
)


def _tc_mid_body(num_ref, den_ref, b_ref, vn_ref, w_ref, asr_ref, adr_ref,
                 h_ref, asad_ref, gm_ref):
    num = num_ref[0] + num_ref[1]
    den = (den_ref[0] + den_ref[1] + 1e-16)[:, None]
    out = num / den + b_ref[...][None, :] + vn_ref[0][None, :]
    rows = lax.broadcasted_iota(jnp.int32, (NPAD, H), 0)
    out = jnp.where(rows < N, out, 0.0)
    h = jnp.dot(out, w_ref[...].T, preferred_element_type=jnp.float32)
    h_ref[...] = h
    a_s = jnp.sum(h * asr_ref[...][None, :], axis=1)
    a_d = jnp.sum(h * adr_ref[...][None, :], axis=1)
    asad_ref[...] = jnp.stack([a_s, a_d])
    m = jnp.max(a_s) + jnp.max(a_d)
    m = jnp.where(m >= 0.0, m, 0.2 * m)
    gm_ref[...] = jnp.full((16,), m, jnp.float32)


_tc_mid = pl.pallas_call(
    _tc_mid_body,
    out_shape=(
        jax.ShapeDtypeStruct((NPAD, H), jnp.float32),
        jax.ShapeDtypeStruct((2, NPAD), jnp.float32),
        jax.ShapeDtypeStruct((16,), jnp.float32),
    ),
)


def _tc_final_body(num_ref, den_ref, b_ref, wo_ref, bo_ref, o_ref):
    num = num_ref[0] + num_ref[1]
    den = (den_ref[0] + den_ref[1] + 1e-16)[:, None]
    out = num / den + b_ref[...][None, :]
    o_ref[...] = (jnp.dot(out, wo_ref[...].T, preferred_element_type=jnp.float32)
                  + bo_ref[...][None, :])


_tc_final = pl.pallas_call(
    _tc_final_body,
    out_shape=jax.ShapeDtypeStruct((NPAD, H), jnp.float32),
)


@functools.partial(
    pl.kernel,
    out_type=(
        jax.ShapeDtypeStruct((NC, NPAD, H), jnp.float32),
        jax.ShapeDtypeStruct((NC, NPAD), jnp.float32),
    ),
    mesh=plsc.VectorSubcoreMesh(core_axis_name="c", subcore_axis_name="s",
                                num_cores=NC, num_subcores=NS),
    compiler_params=pltpu.CompilerParams(needs_layout_passes=False),
    scratch_types=[
        pltpu.VMEM((4, CK), jnp.int32),
        pltpu.VMEM((4, CK), jnp.int32),
        pltpu.VMEM((2, CK), jnp.float32),
        pltpu.VMEM((2, CK), jnp.float32),
        pltpu.VMEM((2, CK), jnp.float32),
        pltpu.VMEM((3, CK, H), jnp.float32),
        pltpu.VMEM((16,), jnp.float32),
        pltpu.VMEM_SHARED((NPAD, H), jnp.float32),
        pltpu.VMEM_SHARED((NPAD,), jnp.float32),
        pltpu.VMEM_SHARED((NPAD,), jnp.float32),
        pltpu.VMEM_SHARED((NPAD,), jnp.float32),
        pltpu.SemaphoreType.DMA,
        pltpu.SemaphoreType.DMA,
        pltpu.SemaphoreType.DMA,
        pltpu.SemaphoreType.DMA((2,)),
        pltpu.SemaphoreType.DMA((2,)),
    ],
)
def _sc_edge(h_hbm, asad_hbm, gm_hbm, src_hbm, dst_hbm, z2_hbm, z1_hbm,
             num_out, den_out,
             sidx_v, didx_v, asg_v, adg_v, ex_v, rows_v, gm_v,
             num_sh, den_sh, asrc_sh, adst_sh,
             semr, sema, semi, semn, semd):
    cid = lax.axis_index("c")
    sid = lax.axis_index("s")
    wid = cid * NS + sid
    r0 = sid * RPT
    rsl = pl.ds(r0, RPT)
    pltpu.sync_copy(z2_hbm.at[rsl], num_sh.at[rsl])
    pltpu.sync_copy(z1_hbm.at[rsl], den_sh.at[rsl])
    pltpu.sync_copy(asad_hbm.at[0].at[rsl], asrc_sh.at[rsl])
    pltpu.sync_copy(asad_hbm.at[1].at[rsl], adst_sh.at[rsl])
    pltpu.sync_copy(gm_hbm, gm_v)
    pltpu.sync_copy(src_hbm.at[wid].at[0], sidx_v.at[0])
    pltpu.sync_copy(dst_hbm.at[wid].at[0], didx_v.at[0])
    pltpu.sync_copy(src_hbm.at[wid].at[1], sidx_v.at[1])
    pltpu.sync_copy(dst_hbm.at[wid].at[1], didx_v.at[1])
    plsc.subcore_barrier()
    gmv = gm_v[...]
    pltpu.async_copy(h_hbm.at[sidx_v.at[0]], rows_v.at[0], semr)
    pltpu.async_copy(asrc_sh.at[sidx_v.at[0]], asg_v.at[0], sema)
    pltpu.async_copy(adst_sh.at[didx_v.at[0]], adg_v.at[0], sema)

    def chunk(c, carry):
        b = lax.rem(c, 3)
        b1 = lax.rem(c + 1, 3)
        p = lax.rem(c, 2)
        p1 = lax.rem(c + 1, 2)
        i1 = lax.rem(c + 1, 4)
        i2 = lax.rem(c + 2, 4)
        ic = lax.rem(c, 4)

        @pl.when(c >= 2)
        def _():
            pltpu.make_async_copy(z2_hbm.at[pl.ds(0, CK)], rows_v.at[b1],
                                  semn.at[p]).wait()
            pltpu.make_async_copy(z1_hbm.at[pl.ds(0, CK)], ex_v.at[p],
                                  semd.at[p]).wait()

        @pl.when(c >= 1)
        def _():
            pltpu.make_async_copy(z1_hbm.at[pl.ds(0, CK)], sidx_v.at[i1],
                                  semi).wait()
            pltpu.make_async_copy(z1_hbm.at[pl.ds(0, CK)], didx_v.at[i1],
                                  semi).wait()

        cnx = jnp.minimum(c + 2, CPW - 1)
        pltpu.async_copy(src_hbm.at[wid].at[cnx], sidx_v.at[i2], semi)
        pltpu.async_copy(dst_hbm.at[wid].at[cnx], didx_v.at[i2], semi)

        @pl.when(c + 1 < CPW)
        def _():
            pltpu.async_copy(h_hbm.at[sidx_v.at[i1]], rows_v.at[b1], semr)
            pltpu.async_copy(asrc_sh.at[sidx_v.at[i1]], asg_v.at[p1], sema)
            pltpu.async_copy(adst_sh.at[didx_v.at[i1]], adg_v.at[p1], sema)

        pltpu.make_async_copy(h_hbm.at[pl.ds(0, CK)], rows_v.at[b], semr).wait()
        pltpu.make_async_copy(z1_hbm.at[pl.ds(0, CK)], asg_v.at[p], sema).wait()
        pltpu.make_async_copy(z1_hbm.at[pl.ds(0, CK)], adg_v.at[p], sema).wait()

        for i in range(CK // 16):
            sl = pl.ds(i * 16, 16)
            av = asg_v[p, sl] + adg_v[p, sl]
            av = jnp.where(av >= 0.0, av, av * 0.2)
            ex_v[p, sl] = jnp.exp(av - gmv)

        pltpu.async_copy(ex_v.at[p], den_sh.at[didx_v.at[ic]], semd.at[p],
                         add=True)

        def sgroup(g, _):
            exg = ex_v[p, pl.ds(g * 16, 16)]
            base = g * 16
            for l in range(16):
                s = exg[l]
                for cc in range(H // 16):
                    sl = pl.ds(cc * 16, 16)
                    rows_v[b, base + l, sl] = rows_v[b, base + l, sl] * s
            return 0

        lax.fori_loop(0, CK // 16, sgroup, 0)

        pltpu.async_copy(rows_v.at[b], num_sh.at[didx_v.at[ic]], semn.at[p],
                         add=True)
        return carry

    lax.fori_loop(0, CPW, chunk, 0)

    pltpu.make_async_copy(z2_hbm.at[pl.ds(0, CK)], rows_v.at[0],
                          semn.at[(CPW - 2) % 2]).wait()
    pltpu.make_async_copy(z1_hbm.at[pl.ds(0, CK)], ex_v.at[0],
                          semd.at[(CPW - 2) % 2]).wait()
    pltpu.make_async_copy(z2_hbm.at[pl.ds(0, CK)], rows_v.at[0],
                          semn.at[(CPW - 1) % 2]).wait()
    pltpu.make_async_copy(z1_hbm.at[pl.ds(0, CK)], ex_v.at[0],
                          semd.at[(CPW - 1) % 2]).wait()
    pltpu.make_async_copy(z1_hbm.at[pl.ds(0, CK)], sidx_v.at[0], semi).wait()
    pltpu.make_async_copy(z1_hbm.at[pl.ds(0, CK)], didx_v.at[0], semi).wait()

    plsc.subcore_barrier()
    pltpu.sync_copy(num_sh.at[rsl], num_out.at[cid].at[rsl])
    pltpu.sync_copy(den_sh.at[rsl], den_out.at[cid].at[rsl])


def kernel(x, edge_index, W1, a_src1, a_dst1, b1, W2, a_src2, a_dst2, b2,
           vn_w, Wm1, bm1, Wm2, bm2, Wo, bo):
    loops = jnp.arange(N, dtype=jnp.int32)
    pad_e = EP - E - N
    src = jnp.concatenate(
        [edge_index[0], loops, jnp.zeros((pad_e,), jnp.int32)])
    dst = jnp.concatenate(
        [edge_index[1], loops, jnp.full((pad_e,), N, jnp.int32)])
    src3 = src.reshape(NW, CPW, CK)
    dst3 = dst.reshape(NW, CPW, CK)
    xp = jnp.pad(x, ((0, NPAD - N), (0, 0)))
    z2 = jnp.zeros((NPAD, H), jnp.float32)
    z1 = jnp.zeros((NPAD,), jnp.float32)

    h1, asad1, gm1 = _tc_prep(xp, W1, a_src1, a_dst1)
    num1, den1 = _sc_edge(h1, asad1, gm1, src3, dst3, z2, z1)
    h2, asad2, gm2 = _tc_mid(num1, den1, b1, vn_w, W2, a_src2, a_dst2)
    num2, den2 = _sc_edge(h2, asad2, gm2, src3, dst3, z2, z1)
    out = _tc_final(num2, den2, b2, Wo, bo)
    return out[:N]
